# baseline (speedup 1.0000x reference)
import jax
import jax.numpy as jnp
from jax import lax
from jax.experimental import pallas as pl
from jax.experimental.pallas import tpu as pltpu
from jax.experimental.pallas import tpu_sc as plsc

VOCAB = 1_000_000
D = 64
B = 4096
CTX = 20
CTXP = 24
NCORES = 2
NSUB = 16
NW = NCORES * NSUB
ROWS_PER_W = B * CTX // NW
CHUNK = 80
NCHUNKS = ROWS_PER_W // CHUNK
VB = 32768


def _sc_gather_sum(idx_hbm, tab2_hbm, out_hbm, idxo_v, idxs_v, rows_v, acc_v, sem):
    c = lax.axis_index("c")
    s = lax.axis_index("s")
    wid = s * NCORES + c
    base = wid * ROWS_PER_W
    zero = jnp.zeros((16,), jnp.float32)
    for l in range(CTXP):
        for j in range(128 // 16):
            acc_v[l, pl.ds(j * 16, 16)] = zero

    def chunk_body(ci, carry):
        pltpu.sync_copy(idx_hbm.at[pl.ds(base + ci * CHUNK, CHUNK)], idxo_v)
        for g in range(CHUNK // 16):
            idxs_v[pl.ds(g * 16, 16)] = (
                lax.shift_right_logical(idxo_v[pl.ds(g * 16, 16)], 1)
            )
        pltpu.async_copy(tab2_hbm.at[idxs_v], rows_v, sem).wait()
        iota16 = lax.iota(jnp.int32, 16)
        for g in range(CHUNK // 16):
            vo = idxo_v[pl.ds(g * 16, 16)]
            par = (vo & 1) * 64
            r16 = g * 16 + iota16
            l16 = (g * 16 + iota16) % CTX
            for d in range(D):
                d16 = jnp.full((16,), d, jnp.int32)
                vals = plsc.load_gather(rows_v, [r16, par + d16])
                plsc.addupdate_scatter(acc_v, [l16, d16], vals)
        return carry

    lax.fori_loop(0, NCHUNKS, chunk_body, 0)
    pltpu.sync_copy(acc_v, out_hbm.at[wid])


def _sc_partial_sums(idx_flat, tab2):
    mesh = plsc.VectorSubcoreMesh(core_axis_name="c", subcore_axis_name="s")
    return pl.kernel(
        _sc_gather_sum,
        out_type=jax.ShapeDtypeStruct((NW, CTXP, 128), jnp.float32),
        mesh=mesh,
        scratch_types=[
            pltpu.VMEM((CHUNK,), jnp.int32),
            pltpu.VMEM((CHUNK,), jnp.int32),
            pltpu.VMEM((CHUNK, 128), jnp.float32),
            pltpu.VMEM((CTXP, 128), jnp.float32),
            pltpu.SemaphoreType.DMA,
        ],
        compiler_params=pltpu.CompilerParams(
            use_tc_tiling_on_sc=True, needs_layout_passes=False
        ),
    )(idx_flat, tab2)


def _mm_body(part_ref, wt_ref, b_ref, out_ref):
    combined = jnp.sum(part_ref[...], axis=0)[:CTX, :D] * (1.0 / B)
    out_ref[...] = (
        lax.dot_general(
            combined,
            wt_ref[...],
            (((1,), (0,)), ((), ())),
            preferred_element_type=jnp.float32,
        )
        + b_ref[...]
    )


def _tc_matmul(partials, Wt, b2d):
    return pl.pallas_call(
        _mm_body,
        grid=(pl.cdiv(VOCAB, VB),),
        in_specs=[
            pl.BlockSpec((NW, CTXP, 128), lambda i: (0, 0, 0)),
            pl.BlockSpec((D, VB), lambda i: (0, i)),
            pl.BlockSpec((1, VB), lambda i: (0, i)),
        ],
        out_specs=pl.BlockSpec((CTX, VB), lambda i: (0, i)),
        out_shape=jax.ShapeDtypeStruct((CTX, VOCAB), jnp.float32),
    )(partials, Wt, b2d)


def kernel(context_idxs, emb_table, W, b):
    idx_flat = context_idxs.reshape(-1).astype(jnp.int32)
    tab2 = emb_table.reshape(VOCAB // 2, 2 * D)
    partials = _sc_partial_sums(idx_flat, tab2)
    return _tc_matmul(partials, W.T, b.reshape(1, VOCAB))
